# Initial kernel scaffold; baseline (speedup 1.0000x reference)
import jax
import jax.numpy as jnp
from jax.experimental import pallas as pl


def kernel(node_scores, children, rels, labels, W, V, rel_emb):
    raise NotImplementedError("write your pallas kernel here")


# speedup vs baseline: 35.4681x; 35.4681x over previous
import jax
import jax.numpy as jnp
from jax import lax
from jax.experimental import pallas as pl
from jax.experimental.pallas import tpu as pltpu
from jax.experimental.pallas import tpu_sc as plsc

B, T, C, P, R = 16384, 32, 8, 3, 9
NC, NS, L = 2, 16, 16
NW = NC * NS
ROWS = B // NW
GROUPS = ROWS // L
NSF = ROWS * T * P
REF = 32


def _softplus(x):
  t = jnp.exp(-jnp.abs(x))
  z = t / (t + 2.0)
  w = z * z
  poly = 1.0 + w * (1.0 / 3.0 + w * (1.0 / 5.0 + w * (1.0 / 7.0 + w * (
      1.0 / 9.0 + w * (1.0 / 11.0 + w * (1.0 / 13.0))))))
  return jnp.maximum(x, 0.0) + 2.0 * z * poly


def _body(ns_hbm, ch_hbm, rl_hbm, w_hbm, v_hbm, re_hbm, out_hbm,
          ns_v, ch_v, rl_v, w_v, v_v, re_v, out_v):
  cid = lax.axis_index("c")
  sid = lax.axis_index("s")
  wid = sid * NC + cid
  base = wid * ROWS

  pltpu.sync_copy(ns_hbm.at[pl.ds(base * T * P, NSF)], ns_v)
  pltpu.sync_copy(ch_hbm.at[pl.ds(base, ROWS)], ch_v)
  pltpu.sync_copy(rl_hbm.at[pl.ds(base, ROWS)], rl_v)
  pltpu.sync_copy(w_hbm, w_v)
  pltpu.sync_copy(v_hbm, v_v)
  pltpu.sync_copy(re_hbm, re_v)

  wvec = w_v[...]
  vvec = v_v[...]
  w = [[wvec[q * P + p] for p in range(P)] for q in range(P)]
  v = [[vvec[q * P + p] for p in range(P)] for q in range(P)]

  lane = lax.broadcasted_iota(jnp.int32, (L,), 0)

  def step(g, i):
    rows = g * L + lane
    spl_i = jnp.zeros((L,), jnp.int32) + i
    c = plsc.load_gather(ch_v, [rows, spl_i])
    r = plsc.load_gather(rl_v, [rows, spl_i])
    rowbase = rows * (T * P)
    pbase = rowbase + i * P
    cbase = rowbase + c * P
    prnt = [plsc.load_gather(ns_v, [pbase + p]) for p in range(P)]
    cs = [plsc.load_gather(ns_v, [cbase + p]) for p in range(P)]
    re = [plsc.load_gather(re_v, [r * P + p]) for p in range(P)]
    a = []
    for p in range(P):
      x = re[p]
      for q in range(P):
        x = x + w[q][p] * prnt[q]
        x = x + v[q][p] * cs[q]
      a.append(_softplus(x) + 1e-6)
    inv = 1.0 / (a[0] + a[1] + a[2])
    for p in range(P):
      plsc.store_scatter(ns_v, [pbase + p], a[p] * inv)

  def g_loop(g, carry):
    def i_loop(i, c2):
      step(g, i)
      return c2
    lax.fori_loop(0, T, i_loop, 0)
    rows = g * L + lane
    src = rows * (T * P) + (T - 1) * P
    dst = rows * P
    for p in range(P):
      val = plsc.load_gather(ns_v, [src + p])
      plsc.store_scatter(out_v, [dst + p], val)
    return carry

  lax.fori_loop(0, GROUPS, g_loop, 0)
  pltpu.sync_copy(out_v, out_hbm.at[pl.ds(base * P, ROWS * P)])


@jax.jit
def _run(ns_flat, ch, rl, wf, vf, ref):
  mesh = plsc.VectorSubcoreMesh(core_axis_name="c", subcore_axis_name="s")
  f = pl.kernel(
      _body,
      out_type=jax.ShapeDtypeStruct((B * P,), jnp.float32),
      mesh=mesh,
      scratch_types=[
          pltpu.VMEM((NSF,), jnp.float32),
          pltpu.VMEM((ROWS, T), jnp.int32),
          pltpu.VMEM((ROWS, T), jnp.int32),
          pltpu.VMEM((L,), jnp.float32),
          pltpu.VMEM((L,), jnp.float32),
          pltpu.VMEM((REF,), jnp.float32),
          pltpu.VMEM((ROWS * P,), jnp.float32),
      ],
      compiler_params=pltpu.CompilerParams(
          needs_layout_passes=False, use_tc_tiling_on_sc=False),
  )
  return f(ns_flat, ch, rl, wf, vf, ref)


def kernel(node_scores, children, rels, labels, W, V, rel_emb):
  del labels
  ch = children[:, :, C - 1].astype(jnp.int32)
  rl = rels[:, :, C - 1].astype(jnp.int32)
  wf = jnp.pad(W.reshape(-1), (0, L - P * P))
  vf = jnp.pad(V.reshape(-1), (0, L - P * P))
  ref = jnp.pad(rel_emb.reshape(-1), (0, REF - R * P))
  out = _run(node_scores.reshape(-1), ch, rl, wf, vf, ref)
  return out.reshape(B, P)
